# Initial kernel scaffold; baseline (speedup 1.0000x reference)
import jax
import jax.numpy as jnp
from jax.experimental import pallas as pl


def kernel(input, e_i, e_a, W_nn2_1, b_nn2_1, W_nn2_2, b_nn2_2, W_nn2_3, b_nn2_3, W_node1, b_node1, W_node2, b_node2, Wx, bx, Wh, bh, wc, bgate, W_lin, b_lin):
    raise NotImplementedError("write your pallas kernel here")


# speedup vs baseline: 14.7625x; 14.7625x over previous
import jax
import jax.numpy as jnp
from jax.experimental import pallas as pl
from jax.experimental.pallas import tpu as pltpu

B, S, IN, N = 64, 50, 16, 32
EMB, HID, K, L, E = 64, 64, 3, 2, 256
INT = 8
NB = N * B
R = B * S * N
TR = 2048
NT = R // TR

_HI = jax.lax.Precision.HIGHEST


def _leaky(x):
    return jnp.where(x >= 0, x, 0.01 * x)


def _prep_kernel(row_ref, col_ref, ea_ref, m_ref, m2_ref):
    row = row_ref[...]
    col = col_ref[...]
    ea = ea_ref[...]
    w = jnp.where(row == col, 0.0, ea)
    lan = jax.lax.broadcasted_iota(jnp.int32, (E, N), 1)
    r_one = (row == lan).astype(jnp.float32)
    c_one = (col == lan).astype(jnp.float32)
    deg = jnp.sum(r_one * w, axis=0, keepdims=True)
    dis = jnp.where(deg > 0, jax.lax.rsqrt(jnp.abs(deg) + 1e-30), 0.0)
    dr = jnp.sum(r_one * dis, axis=1, keepdims=True)
    dc = jnp.sum(c_one * dis, axis=1, keepdims=True)
    norm = -dr * w * dc
    m = jax.lax.dot_general(c_one, r_one * norm, (((0,), (0,)), ((), ())),
                            precision=_HI, preferred_element_type=jnp.float32)
    eye = (jax.lax.broadcasted_iota(jnp.int32, (N, N), 0)
           == jax.lax.broadcasted_iota(jnp.int32, (N, N), 1)).astype(jnp.float32)
    m_ref[...] = m
    m2_ref[...] = 2.0 * jnp.dot(m, m, precision=_HI,
                                preferred_element_type=jnp.float32) - eye


def _f1_kernel(x_ref, w1_ref, st_ref):
    i = pl.program_id(0)
    y = jnp.dot(x_ref[...], w1_ref[...], preferred_element_type=jnp.float32)

    @pl.when(i == 0)
    def _():
        st_ref[...] = jnp.zeros_like(st_ref)

    st_ref[0:1, :] += jnp.sum(y, axis=0, keepdims=True)
    st_ref[1:2, :] += jnp.sum(y * y, axis=0, keepdims=True)


def _f2_kernel(x_ref, w1_ref, w2_ref, mu_ref, inv_ref, y2_ref, st_ref):
    i = pl.program_id(0)
    y1 = jnp.dot(x_ref[...], w1_ref[...], preferred_element_type=jnp.float32)
    a1 = _leaky((y1 - mu_ref[...]) * inv_ref[...])
    y2 = jnp.dot(a1, w2_ref[...], preferred_element_type=jnp.float32)
    y2_ref[...] = y2

    @pl.when(i == 0)
    def _():
        st_ref[...] = jnp.zeros_like(st_ref)

    st_ref[0:1, :] += jnp.sum(y2, axis=0, keepdims=True)
    st_ref[1:2, :] += jnp.sum(y2 * y2, axis=0, keepdims=True)


def _f3_kernel(y2_ref, w3_ref, mu_ref, inv_ref, y3_ref, st_ref):
    i = pl.program_id(0)
    a2 = _leaky((y2_ref[...] - mu_ref[...]) * inv_ref[...])
    y3 = jnp.dot(a2, w3_ref[...], preferred_element_type=jnp.float32)
    y3_ref[...] = y3

    @pl.when(i == 0)
    def _():
        st_ref[...] = jnp.zeros_like(st_ref)

    st_ref[0:1, :] += jnp.sum(y3, axis=0, keepdims=True)
    st_ref[1:2, :] += jnp.sum(y3 * y3, axis=0, keepdims=True)


def _f4_kernel(y3_ref, mu_ref, inv_ref, w1_ref, b1_ref, w2_ref, b2_ref, out_ref):
    d = (y3_ref[0] - mu_ref[...]) * inv_ref[...]
    e1 = _leaky(jnp.dot(d, w1_ref[0], preferred_element_type=jnp.float32)
                + b1_ref[0])
    out_ref[0] = (jnp.dot(e1, w2_ref[0], preferred_element_type=jnp.float32)
                  + b2_ref[0])


def _lstm_kernel(emb_ref, m_ref, m2_ref, wx_ref, wh_ref, b_ref, wc_ref,
                 wlin_ref, blin_ref, out_ref, h_scr, c_scr, hm1_scr, hm2_scr):
    t = pl.program_id(0)

    @pl.when(t == 0)
    def _():
        h_scr[...] = jnp.zeros_like(h_scr)
        c_scr[...] = jnp.zeros_like(c_scr)
        hm1_scr[...] = jnp.zeros_like(hm1_scr)
        hm2_scr[...] = jnp.zeros_like(hm2_scr)

    m = m_ref[...]
    m2 = m2_ref[...]

    def mix(mat, v):
        v3 = v.reshape(N, B, EMB)
        r = jax.lax.dot_general(mat, v3, (((1,), (0,)), ((), ())),
                                precision=_HI,
                                preferred_element_type=jnp.float32)
        return r.reshape(NB, EMB)

    x = emb_ref[0]
    x1 = mix(m, x)
    x2 = mix(m2, x)
    for l in range(L):
        h = h_scr[l]
        c = c_scr[l]
        h1 = hm1_scr[l]
        h2 = hm2_scr[l]
        g = (jnp.dot(x, wx_ref[l, 0], preferred_element_type=jnp.float32)
             + jnp.dot(x1, wx_ref[l, 1], preferred_element_type=jnp.float32)
             + jnp.dot(x2, wx_ref[l, 2], preferred_element_type=jnp.float32)
             + jnp.dot(h, wh_ref[l, 0], preferred_element_type=jnp.float32)
             + jnp.dot(h1, wh_ref[l, 1], preferred_element_type=jnp.float32)
             + jnp.dot(h2, wh_ref[l, 2], preferred_element_type=jnp.float32)
             + b_ref[l])
        gi = g[:, 0:HID]
        gf = g[:, HID:2 * HID]
        gt = g[:, 2 * HID:3 * HID]
        go = g[:, 3 * HID:4 * HID]
        ig = jax.nn.sigmoid(gi + wc_ref[l, 0] * c)
        fg = jax.nn.sigmoid(gf + wc_ref[l, 1] * c)
        tg = jnp.tanh(gt)
        cn = fg * c + ig * tg
        og = jax.nn.sigmoid(go + wc_ref[l, 2] * c)
        hn = og * jnp.tanh(cn)
        h_scr[l] = hn
        c_scr[l] = cn
        hn1 = mix(m, hn)
        hn2 = mix(m2, hn)
        hm1_scr[l] = hn1
        hm2_scr[l] = hn2
        x, x1, x2 = hn, hn1, hn2
    out_ref[...] = (jnp.dot(x, wlin_ref[...], preferred_element_type=jnp.float32)
                    + blin_ref[...])


def kernel(input, e_i, e_a, W_nn2_1, b_nn2_1, W_nn2_2, b_nn2_2, W_nn2_3,
           b_nn2_3, W_node1, b_node1, W_node2, b_node2, Wx, bx, Wh, bh, wc,
           bgate, W_lin, b_lin):
    f32 = jnp.float32
    xT = jnp.transpose(input, (3, 0, 1, 2)).reshape(R, IN)

    row = e_i[0, :, 0:1].astype(jnp.int32)
    col = e_i[0, :, 1:2].astype(jnp.int32)
    eac = e_a[0][:, None]

    m, m2 = pl.pallas_call(
        _prep_kernel,
        out_shape=[jax.ShapeDtypeStruct((N, N), f32)] * 2,
    )(row, col, eac)

    seq = pltpu.CompilerParams(dimension_semantics=("arbitrary",))
    st_spec = pl.BlockSpec((2, 256), lambda i: (0, 0))

    s1 = pl.pallas_call(
        _f1_kernel,
        grid=(NT,),
        in_specs=[pl.BlockSpec((TR, IN), lambda i: (i, 0)),
                  pl.BlockSpec((IN, 256), lambda i: (0, 0))],
        out_specs=st_spec,
        out_shape=jax.ShapeDtypeStruct((2, 256), f32),
        compiler_params=seq,
    )(xT, W_nn2_1)
    mu1 = s1[0:1] / R
    inv1 = jax.lax.rsqrt(s1[1:2] / R - mu1 * mu1 + 1e-5)

    y2, s2 = pl.pallas_call(
        _f2_kernel,
        grid=(NT,),
        in_specs=[pl.BlockSpec((TR, IN), lambda i: (i, 0)),
                  pl.BlockSpec((IN, 256), lambda i: (0, 0)),
                  pl.BlockSpec((256, 256), lambda i: (0, 0)),
                  pl.BlockSpec((1, 256), lambda i: (0, 0)),
                  pl.BlockSpec((1, 256), lambda i: (0, 0))],
        out_specs=[pl.BlockSpec((TR, 256), lambda i: (i, 0)), st_spec],
        out_shape=[jax.ShapeDtypeStruct((R, 256), f32),
                   jax.ShapeDtypeStruct((2, 256), f32)],
        compiler_params=seq,
    )(xT, W_nn2_1, W_nn2_2, mu1, inv1)
    mu2 = s2[0:1] / R
    inv2 = jax.lax.rsqrt(s2[1:2] / R - mu2 * mu2 + 1e-5)

    y3, s3 = pl.pallas_call(
        _f3_kernel,
        grid=(NT,),
        in_specs=[pl.BlockSpec((TR, 256), lambda i: (i, 0)),
                  pl.BlockSpec((256, INT), lambda i: (0, 0)),
                  pl.BlockSpec((1, 256), lambda i: (0, 0)),
                  pl.BlockSpec((1, 256), lambda i: (0, 0))],
        out_specs=[pl.BlockSpec((TR, INT), lambda i: (i, 0)),
                   pl.BlockSpec((2, INT), lambda i: (0, 0))],
        out_shape=[jax.ShapeDtypeStruct((R, INT), f32),
                   jax.ShapeDtypeStruct((2, INT), f32)],
        compiler_params=seq,
    )(y2, W_nn2_3, mu2, inv2)
    mu3 = s3[0:1] / R
    inv3 = jax.lax.rsqrt(s3[1:2] / R - mu3 * mu3 + 1e-5)

    y3r = y3.reshape(N, B * S, INT)
    emb4 = pl.pallas_call(
        _f4_kernel,
        grid=(N,),
        in_specs=[pl.BlockSpec((1, B * S, INT), lambda n: (n, 0, 0)),
                  pl.BlockSpec((1, INT), lambda n: (0, 0)),
                  pl.BlockSpec((1, INT), lambda n: (0, 0)),
                  pl.BlockSpec((1, INT, 256), lambda n: (n, 0, 0)),
                  pl.BlockSpec((1, 1, 256), lambda n: (n, 0, 0)),
                  pl.BlockSpec((1, 256, EMB), lambda n: (n, 0, 0)),
                  pl.BlockSpec((1, 1, EMB), lambda n: (n, 0, 0))],
        out_specs=pl.BlockSpec((1, B * S, EMB), lambda n: (n, 0, 0)),
        out_shape=jax.ShapeDtypeStruct((N, B * S, EMB), f32),
        compiler_params=pltpu.CompilerParams(
            dimension_semantics=("arbitrary",)),
    )(y3r, mu3, inv3, W_node1, b_node1[:, None, :], W_node2,
      b_node2[:, None, :])

    embT = (emb4.reshape(N, B, S, EMB).transpose(2, 0, 1, 3)
            .reshape(S, NB, EMB))
    wxT = Wx.transpose(0, 2, 3, 1, 4).reshape(L, K, EMB, 4 * HID)
    whT = Wh.transpose(0, 2, 3, 1, 4).reshape(L, K, HID, 4 * HID)
    bsum = (bx + bh + bgate).reshape(L, 4 * HID)
    blin = b_lin.reshape(1, 1)

    out2 = pl.pallas_call(
        _lstm_kernel,
        grid=(S,),
        in_specs=[pl.BlockSpec((1, NB, EMB), lambda t: (t, 0, 0)),
                  pl.BlockSpec((N, N), lambda t: (0, 0)),
                  pl.BlockSpec((N, N), lambda t: (0, 0)),
                  pl.BlockSpec((L, K, EMB, 4 * HID), lambda t: (0, 0, 0, 0)),
                  pl.BlockSpec((L, K, HID, 4 * HID), lambda t: (0, 0, 0, 0)),
                  pl.BlockSpec((L, 4 * HID), lambda t: (0, 0)),
                  pl.BlockSpec((L, 3, HID), lambda t: (0, 0, 0)),
                  pl.BlockSpec((HID, 1), lambda t: (0, 0)),
                  pl.BlockSpec((1, 1), lambda t: (0, 0))],
        out_specs=pl.BlockSpec((NB, 1), lambda t: (0, 0)),
        out_shape=jax.ShapeDtypeStruct((NB, 1), f32),
        scratch_shapes=[pltpu.VMEM((L, NB, EMB), f32)] * 4,
        compiler_params=pltpu.CompilerParams(
            dimension_semantics=("arbitrary",)),
    )(embT, m, m2, wxT, whT, bsum, wc, W_lin, blin)

    return out2.reshape(N, B, 1).transpose(1, 0, 2)


# speedup vs baseline: 23.1452x; 1.5678x over previous
import jax
import jax.numpy as jnp
from jax.experimental import pallas as pl
from jax.experimental.pallas import tpu as pltpu

B, S, IN, N = 64, 50, 16, 32
EMB, HID, K, L, E = 64, 64, 3, 2, 256
INT = 8
NB = N * B
R = B * S * N
TR = 2048
NT = R // TR

_HI = jax.lax.Precision.HIGHEST


def _leaky(x):
    return jnp.where(x >= 0, x, 0.01 * x)


def _prep_kernel(row_ref, col_ref, ea_ref, m_ref, m2_ref):
    row = row_ref[...]
    col = col_ref[...]
    ea = ea_ref[...]
    w = jnp.where(row == col, 0.0, ea)
    lan = jax.lax.broadcasted_iota(jnp.int32, (E, N), 1)
    r_one = (row == lan).astype(jnp.float32)
    c_one = (col == lan).astype(jnp.float32)
    deg = jnp.sum(r_one * w, axis=0, keepdims=True)
    dis = jnp.where(deg > 0, jax.lax.rsqrt(jnp.abs(deg) + 1e-30), 0.0)
    dr = jnp.sum(r_one * dis, axis=1, keepdims=True)
    dc = jnp.sum(c_one * dis, axis=1, keepdims=True)
    norm = -dr * w * dc
    m = jax.lax.dot_general(c_one, r_one * norm, (((0,), (0,)), ((), ())),
                            precision=_HI, preferred_element_type=jnp.float32)
    eye = (jax.lax.broadcasted_iota(jnp.int32, (N, N), 0)
           == jax.lax.broadcasted_iota(jnp.int32, (N, N), 1)).astype(jnp.float32)
    m_ref[...] = m
    m2_ref[...] = 2.0 * jnp.dot(m, m, precision=_HI,
                                preferred_element_type=jnp.float32) - eye


def _f1_kernel(x_ref, w1_ref, st_ref):
    i = pl.program_id(0)
    y = jnp.dot(x_ref[...], w1_ref[...], preferred_element_type=jnp.float32)

    @pl.when(i == 0)
    def _():
        st_ref[...] = jnp.zeros_like(st_ref)

    st_ref[0:1, :] += jnp.sum(y, axis=0, keepdims=True)
    st_ref[1:2, :] += jnp.sum(y * y, axis=0, keepdims=True)


def _f2_kernel(x_ref, w1_ref, w2_ref, mu_ref, inv_ref, y2_ref, st_ref):
    i = pl.program_id(0)
    y1 = jnp.dot(x_ref[...], w1_ref[...], preferred_element_type=jnp.float32)
    a1 = _leaky((y1 - mu_ref[...]) * inv_ref[...])
    y2 = jnp.dot(a1, w2_ref[...], preferred_element_type=jnp.float32)
    y2_ref[...] = y2

    @pl.when(i == 0)
    def _():
        st_ref[...] = jnp.zeros_like(st_ref)

    st_ref[0:1, :] += jnp.sum(y2, axis=0, keepdims=True)
    st_ref[1:2, :] += jnp.sum(y2 * y2, axis=0, keepdims=True)


def _f3_kernel(y2_ref, w3_ref, mu_ref, inv_ref, y3_ref, st_ref):
    i = pl.program_id(0)
    a2 = _leaky((y2_ref[...] - mu_ref[...]) * inv_ref[...])
    y3 = jnp.dot(a2, w3_ref[...], preferred_element_type=jnp.float32)
    y3_ref[...] = y3

    @pl.when(i == 0)
    def _():
        st_ref[...] = jnp.zeros_like(st_ref)

    st_ref[0:1, :] += jnp.sum(y3, axis=0, keepdims=True)
    st_ref[1:2, :] += jnp.sum(y3 * y3, axis=0, keepdims=True)


def _f4_kernel(y3_ref, mu_ref, inv_ref, w1_ref, b1_ref, w2_ref, b2_ref, out_ref):
    d = ((y3_ref[:, 0].reshape(S * B, INT) - mu_ref[...]) * inv_ref[...])
    e1 = _leaky(jnp.dot(d, w1_ref[0], preferred_element_type=jnp.float32)
                + b1_ref[0])
    e2 = (jnp.dot(e1, w2_ref[0], preferred_element_type=jnp.float32)
          + b2_ref[0])
    out_ref[:, 0] = e2.reshape(S, B, EMB)


def _lstm_kernel(emb_ref, mcat_ref, wx_ref, wh_ref, b_ref, wc_ref,
                 wlin_ref, blin_ref, out_ref, h_scr, c_scr, hm1_scr, hm2_scr):
    t = pl.program_id(0)

    @pl.when(t == 0)
    def _():
        h_scr[...] = jnp.zeros_like(h_scr)
        c_scr[...] = jnp.zeros_like(c_scr)
        hm1_scr[...] = jnp.zeros_like(hm1_scr)
        hm2_scr[...] = jnp.zeros_like(hm2_scr)

    mcat = mcat_ref[...]

    def mix2(v):
        v3 = v.reshape(N, B, EMB)
        r = jax.lax.dot_general(mcat, v3, (((1,), (0,)), ((), ())),
                                preferred_element_type=jnp.float32)
        r2 = r.reshape(2 * NB, EMB)
        return r2[:NB], r2[NB:]

    x = emb_ref[0]
    x1, x2 = mix2(x)
    for l in range(L):
        h = h_scr[l]
        c = c_scr[l]
        h1 = hm1_scr[l]
        h2 = hm2_scr[l]
        g = (jnp.dot(x, wx_ref[l, 0], preferred_element_type=jnp.float32)
             + jnp.dot(x1, wx_ref[l, 1], preferred_element_type=jnp.float32)
             + jnp.dot(x2, wx_ref[l, 2], preferred_element_type=jnp.float32)
             + jnp.dot(h, wh_ref[l, 0], preferred_element_type=jnp.float32)
             + jnp.dot(h1, wh_ref[l, 1], preferred_element_type=jnp.float32)
             + jnp.dot(h2, wh_ref[l, 2], preferred_element_type=jnp.float32)
             + b_ref[l])
        gi = g[:, 0:HID]
        gf = g[:, HID:2 * HID]
        gt = g[:, 2 * HID:3 * HID]
        go = g[:, 3 * HID:4 * HID]
        ig = jax.nn.sigmoid(gi + wc_ref[l, 0] * c)
        fg = jax.nn.sigmoid(gf + wc_ref[l, 1] * c)
        tg = jnp.tanh(gt)
        cn = fg * c + ig * tg
        og = jax.nn.sigmoid(go + wc_ref[l, 2] * c)
        hn = og * jnp.tanh(cn)
        h_scr[l] = hn
        c_scr[l] = cn
        hn1, hn2 = mix2(hn)
        hm1_scr[l] = hn1
        hm2_scr[l] = hn2
        x, x1, x2 = hn, hn1, hn2
    out_ref[...] = (jnp.dot(x, wlin_ref[...], preferred_element_type=jnp.float32)
                    + blin_ref[...])


def kernel(input, e_i, e_a, W_nn2_1, b_nn2_1, W_nn2_2, b_nn2_2, W_nn2_3,
           b_nn2_3, W_node1, b_node1, W_node2, b_node2, Wx, bx, Wh, bh, wc,
           bgate, W_lin, b_lin):
    f32 = jnp.float32
    xT = jnp.transpose(input, (1, 3, 0, 2)).reshape(R, IN)

    row = e_i[0, :, 0:1].astype(jnp.int32)
    col = e_i[0, :, 1:2].astype(jnp.int32)
    eac = e_a[0][:, None]

    m, m2 = pl.pallas_call(
        _prep_kernel,
        out_shape=[jax.ShapeDtypeStruct((N, N), f32)] * 2,
    )(row, col, eac)

    seq = pltpu.CompilerParams(dimension_semantics=("arbitrary",))
    st_spec = pl.BlockSpec((2, 256), lambda i: (0, 0))

    s1 = pl.pallas_call(
        _f1_kernel,
        grid=(NT,),
        in_specs=[pl.BlockSpec((TR, IN), lambda i: (i, 0)),
                  pl.BlockSpec((IN, 256), lambda i: (0, 0))],
        out_specs=st_spec,
        out_shape=jax.ShapeDtypeStruct((2, 256), f32),
        compiler_params=seq,
    )(xT, W_nn2_1)
    mu1 = s1[0:1] / R
    inv1 = jax.lax.rsqrt(s1[1:2] / R - mu1 * mu1 + 1e-5)

    y2, s2 = pl.pallas_call(
        _f2_kernel,
        grid=(NT,),
        in_specs=[pl.BlockSpec((TR, IN), lambda i: (i, 0)),
                  pl.BlockSpec((IN, 256), lambda i: (0, 0)),
                  pl.BlockSpec((256, 256), lambda i: (0, 0)),
                  pl.BlockSpec((1, 256), lambda i: (0, 0)),
                  pl.BlockSpec((1, 256), lambda i: (0, 0))],
        out_specs=[pl.BlockSpec((TR, 256), lambda i: (i, 0)), st_spec],
        out_shape=[jax.ShapeDtypeStruct((R, 256), f32),
                   jax.ShapeDtypeStruct((2, 256), f32)],
        compiler_params=seq,
    )(xT, W_nn2_1, W_nn2_2, mu1, inv1)
    mu2 = s2[0:1] / R
    inv2 = jax.lax.rsqrt(s2[1:2] / R - mu2 * mu2 + 1e-5)

    y3, s3 = pl.pallas_call(
        _f3_kernel,
        grid=(NT,),
        in_specs=[pl.BlockSpec((TR, 256), lambda i: (i, 0)),
                  pl.BlockSpec((256, INT), lambda i: (0, 0)),
                  pl.BlockSpec((1, 256), lambda i: (0, 0)),
                  pl.BlockSpec((1, 256), lambda i: (0, 0))],
        out_specs=[pl.BlockSpec((TR, INT), lambda i: (i, 0)),
                   pl.BlockSpec((2, INT), lambda i: (0, 0))],
        out_shape=[jax.ShapeDtypeStruct((R, INT), f32),
                   jax.ShapeDtypeStruct((2, INT), f32)],
        compiler_params=seq,
    )(y2, W_nn2_3, mu2, inv2)
    mu3 = s3[0:1] / R
    inv3 = jax.lax.rsqrt(s3[1:2] / R - mu3 * mu3 + 1e-5)

    y3r = y3.reshape(S, N, B, INT)
    emb4 = pl.pallas_call(
        _f4_kernel,
        grid=(N,),
        in_specs=[pl.BlockSpec((S, 1, B, INT), lambda n: (0, n, 0, 0)),
                  pl.BlockSpec((1, INT), lambda n: (0, 0)),
                  pl.BlockSpec((1, INT), lambda n: (0, 0)),
                  pl.BlockSpec((1, INT, 256), lambda n: (n, 0, 0)),
                  pl.BlockSpec((1, 1, 256), lambda n: (n, 0, 0)),
                  pl.BlockSpec((1, 256, EMB), lambda n: (n, 0, 0)),
                  pl.BlockSpec((1, 1, EMB), lambda n: (n, 0, 0))],
        out_specs=pl.BlockSpec((S, 1, B, EMB), lambda n: (0, n, 0, 0)),
        out_shape=jax.ShapeDtypeStruct((S, N, B, EMB), f32),
        compiler_params=pltpu.CompilerParams(
            dimension_semantics=("arbitrary",)),
    )(y3r, mu3, inv3, W_node1, b_node1[:, None, :], W_node2,
      b_node2[:, None, :])

    embT = emb4.reshape(S, NB, EMB)
    mcat = jnp.concatenate([m, m2], axis=0)
    wxT = Wx.transpose(0, 2, 3, 1, 4).reshape(L, K, EMB, 4 * HID)
    whT = Wh.transpose(0, 2, 3, 1, 4).reshape(L, K, HID, 4 * HID)
    bsum = (bx + bh + bgate).reshape(L, 4 * HID)
    blin = b_lin.reshape(1, 1)

    out2 = pl.pallas_call(
        _lstm_kernel,
        grid=(S,),
        in_specs=[pl.BlockSpec((1, NB, EMB), lambda t: (t, 0, 0)),
                  pl.BlockSpec((2 * N, N), lambda t: (0, 0)),
                  pl.BlockSpec((L, K, EMB, 4 * HID), lambda t: (0, 0, 0, 0)),
                  pl.BlockSpec((L, K, HID, 4 * HID), lambda t: (0, 0, 0, 0)),
                  pl.BlockSpec((L, 4 * HID), lambda t: (0, 0)),
                  pl.BlockSpec((L, 3, HID), lambda t: (0, 0, 0)),
                  pl.BlockSpec((HID, 1), lambda t: (0, 0)),
                  pl.BlockSpec((1, 1), lambda t: (0, 0))],
        out_specs=pl.BlockSpec((NB, 1), lambda t: (0, 0)),
        out_shape=jax.ShapeDtypeStruct((NB, 1), f32),
        scratch_shapes=[pltpu.VMEM((L, NB, EMB), f32)] * 4,
        compiler_params=pltpu.CompilerParams(
            dimension_semantics=("arbitrary",)),
    )(embT, mcat, wxT, whT, bsum, wc, W_lin, blin)

    return out2.reshape(N, B, 1).transpose(1, 0, 2)


# speedup vs baseline: 29.5253x; 1.2757x over previous
import jax
import jax.numpy as jnp
from jax.experimental import pallas as pl
from jax.experimental.pallas import tpu as pltpu

B, S, IN, N = 64, 50, 16, 32
EMB, HID, K, L, E = 64, 64, 3, 2, 256
INT = 8
NB = N * B
R = B * S * N
TR = 2048
NT = R // TR

_HI = jax.lax.Precision.HIGHEST


def _leaky(x):
    return jnp.where(x >= 0, x, 0.01 * x)


def _prep_kernel(row_ref, col_ref, ea_ref, m_ref, m2_ref):
    row = row_ref[...]
    col = col_ref[...]
    ea = ea_ref[...]
    w = jnp.where(row == col, 0.0, ea)
    lan = jax.lax.broadcasted_iota(jnp.int32, (E, N), 1)
    r_one = (row == lan).astype(jnp.float32)
    c_one = (col == lan).astype(jnp.float32)
    deg = jnp.sum(r_one * w, axis=0, keepdims=True)
    dis = jnp.where(deg > 0, jax.lax.rsqrt(jnp.abs(deg) + 1e-30), 0.0)
    dr = jnp.sum(r_one * dis, axis=1, keepdims=True)
    dc = jnp.sum(c_one * dis, axis=1, keepdims=True)
    norm = -dr * w * dc
    m = jax.lax.dot_general(c_one, r_one * norm, (((0,), (0,)), ((), ())),
                            precision=_HI, preferred_element_type=jnp.float32)
    eye = (jax.lax.broadcasted_iota(jnp.int32, (N, N), 0)
           == jax.lax.broadcasted_iota(jnp.int32, (N, N), 1)).astype(jnp.float32)
    m_ref[...] = m
    m2_ref[...] = 2.0 * jnp.dot(m, m, precision=_HI,
                                preferred_element_type=jnp.float32) - eye


def _f1_kernel(x_ref, w1_ref, st_ref):
    i = pl.program_id(0)
    y = jnp.dot(x_ref[...], w1_ref[...], preferred_element_type=jnp.float32)

    @pl.when(i == 0)
    def _():
        st_ref[...] = jnp.zeros_like(st_ref)

    st_ref[0:1, :] += jnp.sum(y, axis=0, keepdims=True)
    st_ref[1:2, :] += jnp.sum(y * y, axis=0, keepdims=True)


def _f2_kernel(x_ref, w1_ref, w2_ref, mu_ref, inv_ref, y2_ref, st_ref):
    i = pl.program_id(0)
    y1 = jnp.dot(x_ref[...], w1_ref[...], preferred_element_type=jnp.float32)
    a1 = _leaky((y1 - mu_ref[...]) * inv_ref[...])
    y2 = jnp.dot(a1, w2_ref[...], preferred_element_type=jnp.float32)
    y2_ref[...] = y2

    @pl.when(i == 0)
    def _():
        st_ref[...] = jnp.zeros_like(st_ref)

    st_ref[0:1, :] += jnp.sum(y2, axis=0, keepdims=True)
    st_ref[1:2, :] += jnp.sum(y2 * y2, axis=0, keepdims=True)


def _f3_kernel(y2_ref, w3_ref, mu_ref, inv_ref, y3_ref, st_ref):
    i = pl.program_id(0)
    a2 = _leaky((y2_ref[...] - mu_ref[...]) * inv_ref[...])
    y3 = jnp.dot(a2, w3_ref[...], preferred_element_type=jnp.float32)
    y3_ref[...] = y3

    @pl.when(i == 0)
    def _():
        st_ref[...] = jnp.zeros_like(st_ref)

    st_ref[0:1, :] += jnp.sum(y3, axis=0, keepdims=True)
    st_ref[1:2, :] += jnp.sum(y3 * y3, axis=0, keepdims=True)


def _f4_kernel(y3_ref, mu_ref, inv_ref, w1_ref, b1_ref, w2_ref, b2_ref, out_ref):
    d = ((y3_ref[:, 0].reshape(S * B, INT) - mu_ref[...]) * inv_ref[...])
    e1 = _leaky(jnp.dot(d, w1_ref[0], preferred_element_type=jnp.float32)
                + b1_ref[0])
    e2 = (jnp.dot(e1, w2_ref[0], preferred_element_type=jnp.float32)
          + b2_ref[0])
    out_ref[:, 0] = e2.reshape(S, B, EMB)


def _lstm_kernel(emb_ref, mcat_ref, wx_ref, wh_ref, b_ref, wc_ref,
                 wlin_ref, blin_ref, out_ref, hc_scr, c_scr):
    t = pl.program_id(0)
    mcat = mcat_ref[...]

    def mixcat(v):
        v3 = v.reshape(N, B, EMB)
        r = jax.lax.dot_general(mcat, v3, (((1,), (0,)), ((), ())),
                                preferred_element_type=jnp.float32)
        r2 = r.reshape(2 * NB, EMB)
        return jnp.concatenate([v, r2[:NB], r2[NB:]], axis=1)

    x = mixcat(emb_ref[0])
    hn = None
    for l in range(L):
        hc = jnp.where(t == 0, 0.0, hc_scr[l])
        c = jnp.where(t == 0, 0.0, c_scr[l])
        g = (jnp.dot(x, wx_ref[l], preferred_element_type=jnp.float32)
             + jnp.dot(hc, wh_ref[l], preferred_element_type=jnp.float32)
             + b_ref[l])
        gi = g[:, 0:HID]
        gf = g[:, HID:2 * HID]
        gt = g[:, 2 * HID:3 * HID]
        go = g[:, 3 * HID:4 * HID]
        ig = jax.nn.sigmoid(gi + wc_ref[l, 0] * c)
        fg = jax.nn.sigmoid(gf + wc_ref[l, 1] * c)
        tg = jnp.tanh(gt)
        cn = fg * c + ig * tg
        og = jax.nn.sigmoid(go + wc_ref[l, 2] * c)
        hn = og * jnp.tanh(cn)
        c_scr[l] = cn
        hcn = mixcat(hn)
        hc_scr[l] = hcn
        x = hcn
    out_ref[...] = (jnp.dot(hn, wlin_ref[...], preferred_element_type=jnp.float32)
                    + blin_ref[...])


def kernel(input, e_i, e_a, W_nn2_1, b_nn2_1, W_nn2_2, b_nn2_2, W_nn2_3,
           b_nn2_3, W_node1, b_node1, W_node2, b_node2, Wx, bx, Wh, bh, wc,
           bgate, W_lin, b_lin):
    f32 = jnp.float32
    xT = jnp.transpose(input, (1, 3, 0, 2)).reshape(R, IN)

    row = e_i[0, :, 0:1].astype(jnp.int32)
    col = e_i[0, :, 1:2].astype(jnp.int32)
    eac = e_a[0][:, None]

    m, m2 = pl.pallas_call(
        _prep_kernel,
        out_shape=[jax.ShapeDtypeStruct((N, N), f32)] * 2,
    )(row, col, eac)

    seq = pltpu.CompilerParams(dimension_semantics=("arbitrary",))
    st_spec = pl.BlockSpec((2, 256), lambda i: (0, 0))

    s1 = pl.pallas_call(
        _f1_kernel,
        grid=(NT,),
        in_specs=[pl.BlockSpec((TR, IN), lambda i: (i, 0)),
                  pl.BlockSpec((IN, 256), lambda i: (0, 0))],
        out_specs=st_spec,
        out_shape=jax.ShapeDtypeStruct((2, 256), f32),
        compiler_params=seq,
    )(xT, W_nn2_1)
    mu1 = s1[0:1] / R
    inv1 = jax.lax.rsqrt(s1[1:2] / R - mu1 * mu1 + 1e-5)

    y2, s2 = pl.pallas_call(
        _f2_kernel,
        grid=(NT,),
        in_specs=[pl.BlockSpec((TR, IN), lambda i: (i, 0)),
                  pl.BlockSpec((IN, 256), lambda i: (0, 0)),
                  pl.BlockSpec((256, 256), lambda i: (0, 0)),
                  pl.BlockSpec((1, 256), lambda i: (0, 0)),
                  pl.BlockSpec((1, 256), lambda i: (0, 0))],
        out_specs=[pl.BlockSpec((TR, 256), lambda i: (i, 0)), st_spec],
        out_shape=[jax.ShapeDtypeStruct((R, 256), f32),
                   jax.ShapeDtypeStruct((2, 256), f32)],
        compiler_params=seq,
    )(xT, W_nn2_1, W_nn2_2, mu1, inv1)
    mu2 = s2[0:1] / R
    inv2 = jax.lax.rsqrt(s2[1:2] / R - mu2 * mu2 + 1e-5)

    y3, s3 = pl.pallas_call(
        _f3_kernel,
        grid=(NT,),
        in_specs=[pl.BlockSpec((TR, 256), lambda i: (i, 0)),
                  pl.BlockSpec((256, INT), lambda i: (0, 0)),
                  pl.BlockSpec((1, 256), lambda i: (0, 0)),
                  pl.BlockSpec((1, 256), lambda i: (0, 0))],
        out_specs=[pl.BlockSpec((TR, INT), lambda i: (i, 0)),
                   pl.BlockSpec((2, INT), lambda i: (0, 0))],
        out_shape=[jax.ShapeDtypeStruct((R, INT), f32),
                   jax.ShapeDtypeStruct((2, INT), f32)],
        compiler_params=seq,
    )(y2, W_nn2_3, mu2, inv2)
    mu3 = s3[0:1] / R
    inv3 = jax.lax.rsqrt(s3[1:2] / R - mu3 * mu3 + 1e-5)

    y3r = y3.reshape(S, N, B, INT)
    emb4 = pl.pallas_call(
        _f4_kernel,
        grid=(N,),
        in_specs=[pl.BlockSpec((S, 1, B, INT), lambda n: (0, n, 0, 0)),
                  pl.BlockSpec((1, INT), lambda n: (0, 0)),
                  pl.BlockSpec((1, INT), lambda n: (0, 0)),
                  pl.BlockSpec((1, INT, 256), lambda n: (n, 0, 0)),
                  pl.BlockSpec((1, 1, 256), lambda n: (n, 0, 0)),
                  pl.BlockSpec((1, 256, EMB), lambda n: (n, 0, 0)),
                  pl.BlockSpec((1, 1, EMB), lambda n: (n, 0, 0))],
        out_specs=pl.BlockSpec((S, 1, B, EMB), lambda n: (0, n, 0, 0)),
        out_shape=jax.ShapeDtypeStruct((S, N, B, EMB), f32),
        compiler_params=pltpu.CompilerParams(
            dimension_semantics=("arbitrary",)),
    )(y3r, mu3, inv3, W_node1, b_node1[:, None, :], W_node2,
      b_node2[:, None, :])

    embT = emb4.reshape(S, NB, EMB)
    mcat = jnp.concatenate([m, m2], axis=0)
    wxT = Wx.transpose(0, 2, 3, 1, 4).reshape(L, K * EMB, 4 * HID)
    whT = Wh.transpose(0, 2, 3, 1, 4).reshape(L, K * HID, 4 * HID)
    bsum = (bx + bh + bgate).reshape(L, 4 * HID)
    blin = b_lin.reshape(1, 1)

    out2 = pl.pallas_call(
        _lstm_kernel,
        grid=(S,),
        in_specs=[pl.BlockSpec((1, NB, EMB), lambda t: (t, 0, 0)),
                  pl.BlockSpec((2 * N, N), lambda t: (0, 0)),
                  pl.BlockSpec((L, K * EMB, 4 * HID), lambda t: (0, 0, 0)),
                  pl.BlockSpec((L, K * HID, 4 * HID), lambda t: (0, 0, 0)),
                  pl.BlockSpec((L, 4 * HID), lambda t: (0, 0)),
                  pl.BlockSpec((L, 3, HID), lambda t: (0, 0, 0)),
                  pl.BlockSpec((HID, 1), lambda t: (0, 0)),
                  pl.BlockSpec((1, 1), lambda t: (0, 0))],
        out_specs=pl.BlockSpec((NB, 1), lambda t: (0, 0)),
        out_shape=jax.ShapeDtypeStruct((NB, 1), f32),
        scratch_shapes=[pltpu.VMEM((L, NB, 3 * EMB), f32),
                        pltpu.VMEM((L, NB, EMB), f32)],
        compiler_params=pltpu.CompilerParams(
            dimension_semantics=("arbitrary",)),
    )(embT, mcat, wxT, whT, bsum, wc, W_lin, blin)

    return out2.reshape(N, B, 1).transpose(1, 0, 2)


# speedup vs baseline: 30.1336x; 1.0206x over previous
import jax
import jax.numpy as jnp
from jax.experimental import pallas as pl
from jax.experimental.pallas import tpu as pltpu

B, S, IN, N = 64, 50, 16, 32
EMB, HID, K, L, E = 64, 64, 3, 2, 256
INT = 8
NB = N * B
R = B * S * N
TR = 2048
NT = R // TR
U = 5

_HI = jax.lax.Precision.HIGHEST


def _leaky(x):
    return jnp.where(x >= 0, x, 0.01 * x)


def _prep_kernel(row_ref, col_ref, ea_ref, m_ref, m2_ref):
    row = row_ref[...]
    col = col_ref[...]
    ea = ea_ref[...]
    w = jnp.where(row == col, 0.0, ea)
    lan = jax.lax.broadcasted_iota(jnp.int32, (E, N), 1)
    r_one = (row == lan).astype(jnp.float32)
    c_one = (col == lan).astype(jnp.float32)
    deg = jnp.sum(r_one * w, axis=0, keepdims=True)
    dis = jnp.where(deg > 0, jax.lax.rsqrt(jnp.abs(deg) + 1e-30), 0.0)
    dr = jnp.sum(r_one * dis, axis=1, keepdims=True)
    dc = jnp.sum(c_one * dis, axis=1, keepdims=True)
    norm = -dr * w * dc
    m = jax.lax.dot_general(c_one, r_one * norm, (((0,), (0,)), ((), ())),
                            precision=_HI, preferred_element_type=jnp.float32)
    eye = (jax.lax.broadcasted_iota(jnp.int32, (N, N), 0)
           == jax.lax.broadcasted_iota(jnp.int32, (N, N), 1)).astype(jnp.float32)
    m_ref[...] = m
    m2_ref[...] = 2.0 * jnp.dot(m, m, precision=_HI,
                                preferred_element_type=jnp.float32) - eye


def _f1_kernel(x_ref, w1_ref, st_ref):
    i = pl.program_id(0)
    y = jnp.dot(x_ref[...], w1_ref[...], preferred_element_type=jnp.float32)

    @pl.when(i == 0)
    def _():
        st_ref[...] = jnp.zeros_like(st_ref)

    st_ref[0:1, :] += jnp.sum(y, axis=0, keepdims=True)
    st_ref[1:2, :] += jnp.sum(y * y, axis=0, keepdims=True)


def _f2_kernel(x_ref, w1_ref, w2_ref, mu_ref, inv_ref, y2_ref, st_ref):
    i = pl.program_id(0)
    y1 = jnp.dot(x_ref[...], w1_ref[...], preferred_element_type=jnp.float32)
    a1 = _leaky((y1 - mu_ref[...]) * inv_ref[...])
    y2 = jnp.dot(a1, w2_ref[...], preferred_element_type=jnp.float32)
    y2_ref[...] = y2

    @pl.when(i == 0)
    def _():
        st_ref[...] = jnp.zeros_like(st_ref)

    st_ref[0:1, :] += jnp.sum(y2, axis=0, keepdims=True)
    st_ref[1:2, :] += jnp.sum(y2 * y2, axis=0, keepdims=True)


def _f3_kernel(y2_ref, w3_ref, mu_ref, inv_ref, y3_ref, st_ref):
    i = pl.program_id(0)
    a2 = _leaky((y2_ref[...] - mu_ref[...]) * inv_ref[...])
    y3 = jnp.dot(a2, w3_ref[...], preferred_element_type=jnp.float32)
    y3_ref[...] = y3

    @pl.when(i == 0)
    def _():
        st_ref[...] = jnp.zeros_like(st_ref)

    st_ref[0:1, :] += jnp.sum(y3, axis=0, keepdims=True)
    st_ref[1:2, :] += jnp.sum(y3 * y3, axis=0, keepdims=True)


def _f4_kernel(y3_ref, mu_ref, inv_ref, w1_ref, b1_ref, w2_ref, b2_ref, out_ref):
    d = ((y3_ref[:, 0].reshape(S * B, INT) - mu_ref[...]) * inv_ref[...])
    e1 = _leaky(jnp.dot(d, w1_ref[0], preferred_element_type=jnp.float32)
                + b1_ref[0])
    e2 = (jnp.dot(e1, w2_ref[0], preferred_element_type=jnp.float32)
          + b2_ref[0])
    out_ref[:, 0] = e2.reshape(S, B, EMB)


def _lstm_kernel(emb_ref, mcat_ref, wx_ref, wh_ref, b_ref, wc_ref,
                 wlin_ref, blin_ref, out_ref, hc_scr, c_scr):
    t = pl.program_id(0)
    mcat = mcat_ref[...]

    def mixcat(v):
        v3 = v.reshape(N, B, EMB)
        r = jax.lax.dot_general(mcat, v3, (((1,), (0,)), ((), ())),
                                preferred_element_type=jnp.float32)
        r2 = r.reshape(2 * NB, EMB)
        return jnp.concatenate([v, r2[:NB], r2[NB:]], axis=1)

    hs = [jnp.where(t == 0, 0.0, hc_scr[l]) for l in range(L)]
    cs = [jnp.where(t == 0, 0.0, c_scr[l]) for l in range(L)]
    hn = None
    for u in range(U):
        x = mixcat(emb_ref[u].reshape(NB, EMB))
        for l in range(L):
            c = cs[l]
            g = (jnp.dot(x, wx_ref[l], preferred_element_type=jnp.float32)
                 + jnp.dot(hs[l], wh_ref[l],
                           preferred_element_type=jnp.float32)
                 + b_ref[l])
            gi = g[:, 0:HID]
            gf = g[:, HID:2 * HID]
            gt = g[:, 2 * HID:3 * HID]
            go = g[:, 3 * HID:4 * HID]
            ig = jax.nn.sigmoid(gi + wc_ref[l, 0] * c)
            fg = jax.nn.sigmoid(gf + wc_ref[l, 1] * c)
            tg = jnp.tanh(gt)
            cn = fg * c + ig * tg
            og = jax.nn.sigmoid(go + wc_ref[l, 2] * c)
            hn = og * jnp.tanh(cn)
            cs[l] = cn
            hs[l] = mixcat(hn)
            x = hs[l]
    for l in range(L):
        hc_scr[l] = hs[l]
        c_scr[l] = cs[l]
    out_ref[...] = (jnp.dot(hn, wlin_ref[...], preferred_element_type=jnp.float32)
                    + blin_ref[...])


def kernel(input, e_i, e_a, W_nn2_1, b_nn2_1, W_nn2_2, b_nn2_2, W_nn2_3,
           b_nn2_3, W_node1, b_node1, W_node2, b_node2, Wx, bx, Wh, bh, wc,
           bgate, W_lin, b_lin):
    f32 = jnp.float32
    xT = jnp.transpose(input, (1, 3, 0, 2)).reshape(R, IN)

    row = e_i[0, :, 0:1].astype(jnp.int32)
    col = e_i[0, :, 1:2].astype(jnp.int32)
    eac = e_a[0][:, None]

    m, m2 = pl.pallas_call(
        _prep_kernel,
        out_shape=[jax.ShapeDtypeStruct((N, N), f32)] * 2,
    )(row, col, eac)

    seq = pltpu.CompilerParams(dimension_semantics=("arbitrary",))
    st_spec = pl.BlockSpec((2, 256), lambda i: (0, 0))

    s1 = pl.pallas_call(
        _f1_kernel,
        grid=(NT,),
        in_specs=[pl.BlockSpec((TR, IN), lambda i: (i, 0)),
                  pl.BlockSpec((IN, 256), lambda i: (0, 0))],
        out_specs=st_spec,
        out_shape=jax.ShapeDtypeStruct((2, 256), f32),
        compiler_params=seq,
    )(xT, W_nn2_1)
    mu1 = s1[0:1] / R
    inv1 = jax.lax.rsqrt(s1[1:2] / R - mu1 * mu1 + 1e-5)

    y2, s2 = pl.pallas_call(
        _f2_kernel,
        grid=(NT,),
        in_specs=[pl.BlockSpec((TR, IN), lambda i: (i, 0)),
                  pl.BlockSpec((IN, 256), lambda i: (0, 0)),
                  pl.BlockSpec((256, 256), lambda i: (0, 0)),
                  pl.BlockSpec((1, 256), lambda i: (0, 0)),
                  pl.BlockSpec((1, 256), lambda i: (0, 0))],
        out_specs=[pl.BlockSpec((TR, 256), lambda i: (i, 0)), st_spec],
        out_shape=[jax.ShapeDtypeStruct((R, 256), f32),
                   jax.ShapeDtypeStruct((2, 256), f32)],
        compiler_params=seq,
    )(xT, W_nn2_1, W_nn2_2, mu1, inv1)
    mu2 = s2[0:1] / R
    inv2 = jax.lax.rsqrt(s2[1:2] / R - mu2 * mu2 + 1e-5)

    y3, s3 = pl.pallas_call(
        _f3_kernel,
        grid=(NT,),
        in_specs=[pl.BlockSpec((TR, 256), lambda i: (i, 0)),
                  pl.BlockSpec((256, INT), lambda i: (0, 0)),
                  pl.BlockSpec((1, 256), lambda i: (0, 0)),
                  pl.BlockSpec((1, 256), lambda i: (0, 0))],
        out_specs=[pl.BlockSpec((TR, INT), lambda i: (i, 0)),
                   pl.BlockSpec((2, INT), lambda i: (0, 0))],
        out_shape=[jax.ShapeDtypeStruct((R, INT), f32),
                   jax.ShapeDtypeStruct((2, INT), f32)],
        compiler_params=seq,
    )(y2, W_nn2_3, mu2, inv2)
    mu3 = s3[0:1] / R
    inv3 = jax.lax.rsqrt(s3[1:2] / R - mu3 * mu3 + 1e-5)

    y3r = y3.reshape(S, N, B, INT)
    emb4 = pl.pallas_call(
        _f4_kernel,
        grid=(N,),
        in_specs=[pl.BlockSpec((S, 1, B, INT), lambda n: (0, n, 0, 0)),
                  pl.BlockSpec((1, INT), lambda n: (0, 0)),
                  pl.BlockSpec((1, INT), lambda n: (0, 0)),
                  pl.BlockSpec((1, INT, 256), lambda n: (n, 0, 0)),
                  pl.BlockSpec((1, 1, 256), lambda n: (n, 0, 0)),
                  pl.BlockSpec((1, 256, EMB), lambda n: (n, 0, 0)),
                  pl.BlockSpec((1, 1, EMB), lambda n: (n, 0, 0))],
        out_specs=pl.BlockSpec((S, 1, B, EMB), lambda n: (0, n, 0, 0)),
        out_shape=jax.ShapeDtypeStruct((S, N, B, EMB), f32),
        compiler_params=pltpu.CompilerParams(
            dimension_semantics=("arbitrary",)),
    )(y3r, mu3, inv3, W_node1, b_node1[:, None, :], W_node2,
      b_node2[:, None, :])

    mcat = jnp.concatenate([m, m2], axis=0)
    wxT = Wx.transpose(0, 2, 3, 1, 4).reshape(L, K * EMB, 4 * HID)
    whT = Wh.transpose(0, 2, 3, 1, 4).reshape(L, K * HID, 4 * HID)
    bsum = (bx + bh + bgate).reshape(L, 4 * HID)
    blin = b_lin.reshape(1, 1)

    out2 = pl.pallas_call(
        _lstm_kernel,
        grid=(S // U,),
        in_specs=[pl.BlockSpec((U, N, B, EMB), lambda t: (t, 0, 0, 0)),
                  pl.BlockSpec((2 * N, N), lambda t: (0, 0)),
                  pl.BlockSpec((L, K * EMB, 4 * HID), lambda t: (0, 0, 0)),
                  pl.BlockSpec((L, K * HID, 4 * HID), lambda t: (0, 0, 0)),
                  pl.BlockSpec((L, 4 * HID), lambda t: (0, 0)),
                  pl.BlockSpec((L, 3, HID), lambda t: (0, 0, 0)),
                  pl.BlockSpec((HID, 1), lambda t: (0, 0)),
                  pl.BlockSpec((1, 1), lambda t: (0, 0))],
        out_specs=pl.BlockSpec((NB, 1), lambda t: (0, 0)),
        out_shape=jax.ShapeDtypeStruct((NB, 1), f32),
        scratch_shapes=[pltpu.VMEM((L, NB, 3 * EMB), f32),
                        pltpu.VMEM((L, NB, EMB), f32)],
        compiler_params=pltpu.CompilerParams(
            dimension_semantics=("arbitrary",)),
    )(emb4, mcat, wxT, whT, bsum, wc, W_lin, blin)

    return out2.reshape(N, B, 1).transpose(1, 0, 2)


# speedup vs baseline: 31.3091x; 1.0390x over previous
import jax
import jax.numpy as jnp
from jax.experimental import pallas as pl
from jax.experimental.pallas import tpu as pltpu

B, S, IN, N = 64, 50, 16, 32
EMB, HID, K, L, E = 64, 64, 3, 2, 256
INT = 8
NB = N * B
R = B * S * N
TR = 2048
NT = R // TR
U = 5

_HI = jax.lax.Precision.HIGHEST


def _leaky(x):
    return jnp.where(x >= 0, x, 0.01 * x)


def _sig(z):
    return 0.5 * jnp.tanh(0.5 * z) + 0.5


def _bn_mu_inv(s_ref):
    mu = s_ref[0:1, :] * (1.0 / R)
    ex2 = s_ref[1:2, :] * (1.0 / R)
    return mu, jax.lax.rsqrt(ex2 - mu * mu + 1e-5)


def _prep_kernel(row_ref, col_ref, ea_ref, m_ref):
    row = row_ref[...]
    col = col_ref[...]
    ea = ea_ref[...]
    w = jnp.where(row == col, 0.0, ea)
    lan = jax.lax.broadcasted_iota(jnp.int32, (E, N), 1)
    r_one = (row == lan).astype(jnp.float32)
    c_one = (col == lan).astype(jnp.float32)
    deg = jnp.sum(r_one * w, axis=0, keepdims=True)
    dis = jnp.where(deg > 0, jax.lax.rsqrt(jnp.abs(deg) + 1e-30), 0.0)
    dr = jnp.sum(r_one * dis, axis=1, keepdims=True)
    dc = jnp.sum(c_one * dis, axis=1, keepdims=True)
    norm = -dr * w * dc
    m = jax.lax.dot_general(c_one, r_one * norm, (((0,), (0,)), ((), ())),
                            precision=_HI, preferred_element_type=jnp.float32)
    eye = (jax.lax.broadcasted_iota(jnp.int32, (N, N), 0)
           == jax.lax.broadcasted_iota(jnp.int32, (N, N), 1)).astype(jnp.float32)
    m_ref[0:N] = m
    m_ref[N:] = 2.0 * jnp.dot(m, m, precision=_HI,
                              preferred_element_type=jnp.float32) - eye


def _f1_kernel(x_ref, w1_ref, st_ref):
    i = pl.program_id(0)
    y = jnp.dot(x_ref[...], w1_ref[...], preferred_element_type=jnp.float32)

    @pl.when(i == 0)
    def _():
        st_ref[...] = jnp.zeros_like(st_ref)

    st_ref[0:1, :] += jnp.sum(y, axis=0, keepdims=True)
    st_ref[1:2, :] += jnp.sum(y * y, axis=0, keepdims=True)


def _f2_kernel(x_ref, w1_ref, w2_ref, s1_ref, y2_ref, st_ref):
    i = pl.program_id(0)
    mu, inv = _bn_mu_inv(s1_ref)
    y1 = jnp.dot(x_ref[...], w1_ref[...], preferred_element_type=jnp.float32)
    a1 = _leaky((y1 - mu) * inv)
    y2 = jnp.dot(a1, w2_ref[...], preferred_element_type=jnp.float32)
    y2_ref[...] = y2.astype(jnp.bfloat16)

    @pl.when(i == 0)
    def _():
        st_ref[...] = jnp.zeros_like(st_ref)

    st_ref[0:1, :] += jnp.sum(y2, axis=0, keepdims=True)
    st_ref[1:2, :] += jnp.sum(y2 * y2, axis=0, keepdims=True)


def _f3_kernel(y2_ref, w3_ref, s2_ref, y3_ref, st_ref):
    i = pl.program_id(0)
    mu, inv = _bn_mu_inv(s2_ref)
    a2 = _leaky((y2_ref[...].astype(jnp.float32) - mu) * inv)
    y3 = jnp.dot(a2, w3_ref[...], preferred_element_type=jnp.float32)
    y3_ref[...] = y3

    @pl.when(i == 0)
    def _():
        st_ref[...] = jnp.zeros_like(st_ref)

    st_ref[0:1, :] += jnp.sum(y3, axis=0, keepdims=True)
    st_ref[1:2, :] += jnp.sum(y3 * y3, axis=0, keepdims=True)


def _f4_kernel(y3_ref, s3_ref, w1_ref, b1_ref, w2_ref, b2_ref, out_ref):
    mu, inv = _bn_mu_inv(s3_ref)
    d = ((y3_ref[:, 0].reshape(S * B, INT) - mu) * inv)
    e1 = _leaky(jnp.dot(d, w1_ref[0], preferred_element_type=jnp.float32)
                + b1_ref[0])
    e2 = (jnp.dot(e1, w2_ref[0], preferred_element_type=jnp.float32)
          + b2_ref[0])
    out_ref[:, 0] = e2.reshape(S, B, EMB)


def _lstm_kernel(emb_ref, mcat_ref, wx_ref, wh_ref, b_ref, wc_ref,
                 wlin_ref, blin_ref, out_ref, hc_scr, c_scr):
    t = pl.program_id(0)
    mcat = mcat_ref[...]

    def mixcat(v):
        v3 = v.reshape(N, B, EMB)
        r = jax.lax.dot_general(mcat, v3, (((1,), (0,)), ((), ())),
                                preferred_element_type=jnp.float32)
        r2 = r.reshape(2 * NB, EMB)
        return jnp.concatenate([v, r2[:NB], r2[NB:]], axis=1)

    hs = [jnp.where(t == 0, 0.0, hc_scr[l]) for l in range(L)]
    cs = [jnp.where(t == 0, 0.0, c_scr[l]) for l in range(L)]
    hn = None
    for u in range(U):
        x = mixcat(emb_ref[u].reshape(NB, EMB))
        for l in range(L):
            c = cs[l]
            g = (jnp.dot(x, wx_ref[l], preferred_element_type=jnp.float32)
                 + jnp.dot(hs[l], wh_ref[l],
                           preferred_element_type=jnp.float32)
                 + b_ref[l])
            gi = g[:, 0:HID]
            gf = g[:, HID:2 * HID]
            gt = g[:, 2 * HID:3 * HID]
            go = g[:, 3 * HID:4 * HID]
            ig = _sig(gi + wc_ref[l, 0] * c)
            fg = _sig(gf + wc_ref[l, 1] * c)
            tg = jnp.tanh(gt)
            cn = fg * c + ig * tg
            og = _sig(go + wc_ref[l, 2] * c)
            hn = og * jnp.tanh(cn)
            cs[l] = cn
            hs[l] = mixcat(hn)
            x = hs[l]
    for l in range(L):
        hc_scr[l] = hs[l]
        c_scr[l] = cs[l]
    out_ref[...] = (jnp.dot(hn, wlin_ref[...], preferred_element_type=jnp.float32)
                    + blin_ref[...])


def kernel(input, e_i, e_a, W_nn2_1, b_nn2_1, W_nn2_2, b_nn2_2, W_nn2_3,
           b_nn2_3, W_node1, b_node1, W_node2, b_node2, Wx, bx, Wh, bh, wc,
           bgate, W_lin, b_lin):
    f32 = jnp.float32
    xT = jnp.transpose(input, (1, 3, 0, 2)).reshape(R, IN)

    row = e_i[0, :, 0:1].astype(jnp.int32)
    col = e_i[0, :, 1:2].astype(jnp.int32)
    eac = e_a[0][:, None]

    mcat = pl.pallas_call(
        _prep_kernel,
        out_shape=jax.ShapeDtypeStruct((2 * N, N), f32),
    )(row, col, eac)

    seq = pltpu.CompilerParams(dimension_semantics=("arbitrary",))
    st_spec = pl.BlockSpec((2, 256), lambda i: (0, 0))

    s1 = pl.pallas_call(
        _f1_kernel,
        grid=(NT,),
        in_specs=[pl.BlockSpec((TR, IN), lambda i: (i, 0)),
                  pl.BlockSpec((IN, 256), lambda i: (0, 0))],
        out_specs=st_spec,
        out_shape=jax.ShapeDtypeStruct((2, 256), f32),
        compiler_params=seq,
    )(xT, W_nn2_1)

    y2, s2 = pl.pallas_call(
        _f2_kernel,
        grid=(NT,),
        in_specs=[pl.BlockSpec((TR, IN), lambda i: (i, 0)),
                  pl.BlockSpec((IN, 256), lambda i: (0, 0)),
                  pl.BlockSpec((256, 256), lambda i: (0, 0)),
                  pl.BlockSpec((2, 256), lambda i: (0, 0))],
        out_specs=[pl.BlockSpec((TR, 256), lambda i: (i, 0)), st_spec],
        out_shape=[jax.ShapeDtypeStruct((R, 256), jnp.bfloat16),
                   jax.ShapeDtypeStruct((2, 256), f32)],
        compiler_params=seq,
    )(xT, W_nn2_1, W_nn2_2, s1)

    y3, s3 = pl.pallas_call(
        _f3_kernel,
        grid=(NT,),
        in_specs=[pl.BlockSpec((TR, 256), lambda i: (i, 0)),
                  pl.BlockSpec((256, INT), lambda i: (0, 0)),
                  pl.BlockSpec((2, 256), lambda i: (0, 0))],
        out_specs=[pl.BlockSpec((TR, INT), lambda i: (i, 0)),
                   pl.BlockSpec((2, INT), lambda i: (0, 0))],
        out_shape=[jax.ShapeDtypeStruct((R, INT), f32),
                   jax.ShapeDtypeStruct((2, INT), f32)],
        compiler_params=seq,
    )(y2, W_nn2_3, s2)

    y3r = y3.reshape(S, N, B, INT)
    emb4 = pl.pallas_call(
        _f4_kernel,
        grid=(N,),
        in_specs=[pl.BlockSpec((S, 1, B, INT), lambda n: (0, n, 0, 0)),
                  pl.BlockSpec((2, INT), lambda n: (0, 0)),
                  pl.BlockSpec((1, INT, 256), lambda n: (n, 0, 0)),
                  pl.BlockSpec((1, 1, 256), lambda n: (n, 0, 0)),
                  pl.BlockSpec((1, 256, EMB), lambda n: (n, 0, 0)),
                  pl.BlockSpec((1, 1, EMB), lambda n: (n, 0, 0))],
        out_specs=pl.BlockSpec((S, 1, B, EMB), lambda n: (0, n, 0, 0)),
        out_shape=jax.ShapeDtypeStruct((S, N, B, EMB), f32),
        compiler_params=pltpu.CompilerParams(
            dimension_semantics=("arbitrary",)),
    )(y3r, s3, W_node1, b_node1[:, None, :], W_node2,
      b_node2[:, None, :])

    wxT = Wx.transpose(0, 2, 3, 1, 4).reshape(L, K * EMB, 4 * HID)
    whT = Wh.transpose(0, 2, 3, 1, 4).reshape(L, K * HID, 4 * HID)
    bsum = (bx + bh + bgate).reshape(L, 4 * HID)
    blin = b_lin.reshape(1, 1)

    out2 = pl.pallas_call(
        _lstm_kernel,
        grid=(S // U,),
        in_specs=[pl.BlockSpec((U, N, B, EMB), lambda t: (t, 0, 0, 0)),
                  pl.BlockSpec((2 * N, N), lambda t: (0, 0)),
                  pl.BlockSpec((L, K * EMB, 4 * HID), lambda t: (0, 0, 0)),
                  pl.BlockSpec((L, K * HID, 4 * HID), lambda t: (0, 0, 0)),
                  pl.BlockSpec((L, 4 * HID), lambda t: (0, 0)),
                  pl.BlockSpec((L, 3, HID), lambda t: (0, 0, 0)),
                  pl.BlockSpec((HID, 1), lambda t: (0, 0)),
                  pl.BlockSpec((1, 1), lambda t: (0, 0))],
        out_specs=pl.BlockSpec((NB, 1), lambda t: (0, 0)),
        out_shape=jax.ShapeDtypeStruct((NB, 1), f32),
        scratch_shapes=[pltpu.VMEM((L, NB, 3 * EMB), f32),
                        pltpu.VMEM((L, NB, EMB), f32)],
        compiler_params=pltpu.CompilerParams(
            dimension_semantics=("arbitrary",)),
    )(emb4, mcat, wxT, whT, bsum, wc, W_lin, blin)

    return out2.reshape(N, B, 1).transpose(1, 0, 2)


# speedup vs baseline: 33.9505x; 1.0844x over previous
import jax
import jax.numpy as jnp
from jax.experimental import pallas as pl
from jax.experimental.pallas import tpu as pltpu

B, S, IN, N = 64, 50, 16, 32
EMB, HID, K, L, E = 64, 64, 3, 2, 256
INT = 8
NB = N * B
R = B * S * N
TR = 4096
NT = R // TR
U = 5

_HI = jax.lax.Precision.HIGHEST


def _leaky(x):
    return jnp.where(x >= 0, x, 0.01 * x)


def _sig(z):
    return 0.5 * jnp.tanh(0.5 * z) + 0.5


def _bn_mu_inv(s_ref):
    mu = s_ref[0:1, :] * (1.0 / R)
    ex2 = s_ref[1:2, :] * (1.0 / R)
    return mu, jax.lax.rsqrt(ex2 - mu * mu + 1e-5)


def _prep_kernel(row_ref, col_ref, ea_ref, m_ref):
    row = row_ref[...]
    col = col_ref[...]
    ea = ea_ref[...]
    w = jnp.where(row == col, 0.0, ea)
    lan = jax.lax.broadcasted_iota(jnp.int32, (E, N), 1)
    r_one = (row == lan).astype(jnp.float32)
    c_one = (col == lan).astype(jnp.float32)
    deg = jnp.sum(r_one * w, axis=0, keepdims=True)
    dis = jnp.where(deg > 0, jax.lax.rsqrt(jnp.abs(deg) + 1e-30), 0.0)
    dr = jnp.sum(r_one * dis, axis=1, keepdims=True)
    dc = jnp.sum(c_one * dis, axis=1, keepdims=True)
    norm = -dr * w * dc
    m = jax.lax.dot_general(c_one, r_one * norm, (((0,), (0,)), ((), ())),
                            precision=_HI, preferred_element_type=jnp.float32)
    eye = (jax.lax.broadcasted_iota(jnp.int32, (N, N), 0)
           == jax.lax.broadcasted_iota(jnp.int32, (N, N), 1)).astype(jnp.float32)
    m_ref[0:N] = m
    m_ref[N:] = 2.0 * jnp.dot(m, m, precision=_HI,
                              preferred_element_type=jnp.float32) - eye


def _f1_kernel(x_ref, g_ref, cs_ref):
    i = pl.program_id(0)
    x = x_ref[...]

    @pl.when(i == 0)
    def _():
        g_ref[...] = jnp.zeros_like(g_ref)
        cs_ref[...] = jnp.zeros_like(cs_ref)

    g_ref[...] += jax.lax.dot_general(x, x, (((0,), (0,)), ((), ())),
                                      precision=_HI,
                                      preferred_element_type=jnp.float32)
    cs_ref[...] += jnp.sum(x, axis=0, keepdims=True)


def _f2_kernel(x_ref, w1_ref, w2_ref, g_ref, cs_ref, y2_ref, st_ref):
    i = pl.program_id(0)
    w1 = w1_ref[...]
    mu = jnp.dot(cs_ref[...], w1, precision=_HI,
                 preferred_element_type=jnp.float32) * (1.0 / R)
    ex2 = jnp.sum(w1 * jnp.dot(g_ref[...], w1, precision=_HI,
                               preferred_element_type=jnp.float32),
                  axis=0, keepdims=True) * (1.0 / R)
    inv = jax.lax.rsqrt(ex2 - mu * mu + 1e-5)
    y1 = jnp.dot(x_ref[...], w1, preferred_element_type=jnp.float32)
    a1 = _leaky((y1 - mu) * inv)
    y2 = jnp.dot(a1, w2_ref[...], preferred_element_type=jnp.float32)
    y2_ref[...] = y2.astype(jnp.bfloat16)

    @pl.when(i == 0)
    def _():
        st_ref[...] = jnp.zeros_like(st_ref)

    st_ref[0:1, :] += jnp.sum(y2, axis=0, keepdims=True)
    st_ref[1:2, :] += jnp.sum(y2 * y2, axis=0, keepdims=True)


def _f3_kernel(y2_ref, w3_ref, s2_ref, y3_ref, st_ref):
    i = pl.program_id(0)
    mu, inv = _bn_mu_inv(s2_ref)
    a2 = _leaky((y2_ref[...].astype(jnp.float32) - mu) * inv)
    y3 = jnp.dot(a2, w3_ref[...], preferred_element_type=jnp.float32)
    y3_ref[...] = y3

    @pl.when(i == 0)
    def _():
        st_ref[...] = jnp.zeros_like(st_ref)

    st_ref[0:1, :] += jnp.sum(y3, axis=0, keepdims=True)
    st_ref[1:2, :] += jnp.sum(y3 * y3, axis=0, keepdims=True)


def _f4_kernel(y3_ref, s3_ref, w1_ref, b1_ref, w2_ref, b2_ref, out_ref):
    mu, inv = _bn_mu_inv(s3_ref)
    d = ((y3_ref[:, 0].reshape(S * B, INT) - mu) * inv)
    e1 = _leaky(jnp.dot(d, w1_ref[0], preferred_element_type=jnp.float32)
                + b1_ref[0])
    e2 = (jnp.dot(e1, w2_ref[0], preferred_element_type=jnp.float32)
          + b2_ref[0])
    out_ref[:, 0] = e2.reshape(S, B, EMB)


def _lstm_kernel(emb_ref, mcat_ref, wx_ref, wh_ref, b_ref, wc_ref,
                 wlin_ref, blin_ref, out_ref, hc_scr, c_scr):
    t = pl.program_id(0)
    mcat = mcat_ref[...]

    def mixcat(v):
        v3 = v.reshape(N, B, EMB)
        r = jax.lax.dot_general(mcat, v3, (((1,), (0,)), ((), ())),
                                preferred_element_type=jnp.float32)
        r2 = r.reshape(2 * NB, EMB)
        return jnp.concatenate([v, r2[:NB], r2[NB:]], axis=1)

    hs = [jnp.where(t == 0, 0.0, hc_scr[l]) for l in range(L)]
    cs = [jnp.where(t == 0, 0.0, c_scr[l]) for l in range(L)]
    hn = None
    for u in range(U):
        x = mixcat(emb_ref[u].reshape(NB, EMB))
        for l in range(L):
            c = cs[l]
            g = (jnp.dot(x, wx_ref[l], preferred_element_type=jnp.float32)
                 + jnp.dot(hs[l], wh_ref[l],
                           preferred_element_type=jnp.float32)
                 + b_ref[l])
            gi = g[:, 0:HID]
            gf = g[:, HID:2 * HID]
            gt = g[:, 2 * HID:3 * HID]
            go = g[:, 3 * HID:4 * HID]
            ig = _sig(gi + wc_ref[l, 0] * c)
            fg = _sig(gf + wc_ref[l, 1] * c)
            tg = jnp.tanh(gt)
            cn = fg * c + ig * tg
            og = _sig(go + wc_ref[l, 2] * c)
            hn = og * jnp.tanh(cn)
            cs[l] = cn
            hs[l] = mixcat(hn)
            x = hs[l]
    for l in range(L):
        hc_scr[l] = hs[l]
        c_scr[l] = cs[l]
    out_ref[...] = (jnp.dot(hn, wlin_ref[...], preferred_element_type=jnp.float32)
                    + blin_ref[...])


def kernel(input, e_i, e_a, W_nn2_1, b_nn2_1, W_nn2_2, b_nn2_2, W_nn2_3,
           b_nn2_3, W_node1, b_node1, W_node2, b_node2, Wx, bx, Wh, bh, wc,
           bgate, W_lin, b_lin):
    f32 = jnp.float32
    xT = jnp.transpose(input, (1, 3, 0, 2)).reshape(R, IN)

    row = e_i[0, :, 0:1].astype(jnp.int32)
    col = e_i[0, :, 1:2].astype(jnp.int32)
    eac = e_a[0][:, None]

    mcat = pl.pallas_call(
        _prep_kernel,
        out_shape=jax.ShapeDtypeStruct((2 * N, N), f32),
    )(row, col, eac)

    seq = pltpu.CompilerParams(dimension_semantics=("arbitrary",))
    st_spec = pl.BlockSpec((2, 256), lambda i: (0, 0))

    g1, cs1 = pl.pallas_call(
        _f1_kernel,
        grid=(NT,),
        in_specs=[pl.BlockSpec((TR, IN), lambda i: (i, 0))],
        out_specs=[pl.BlockSpec((IN, IN), lambda i: (0, 0)),
                   pl.BlockSpec((1, IN), lambda i: (0, 0))],
        out_shape=[jax.ShapeDtypeStruct((IN, IN), f32),
                   jax.ShapeDtypeStruct((1, IN), f32)],
        compiler_params=seq,
    )(xT)

    y2, s2 = pl.pallas_call(
        _f2_kernel,
        grid=(NT,),
        in_specs=[pl.BlockSpec((TR, IN), lambda i: (i, 0)),
                  pl.BlockSpec((IN, 256), lambda i: (0, 0)),
                  pl.BlockSpec((256, 256), lambda i: (0, 0)),
                  pl.BlockSpec((IN, IN), lambda i: (0, 0)),
                  pl.BlockSpec((1, IN), lambda i: (0, 0))],
        out_specs=[pl.BlockSpec((TR, 256), lambda i: (i, 0)), st_spec],
        out_shape=[jax.ShapeDtypeStruct((R, 256), jnp.bfloat16),
                   jax.ShapeDtypeStruct((2, 256), f32)],
        compiler_params=seq,
    )(xT, W_nn2_1, W_nn2_2, g1, cs1)

    y3, s3 = pl.pallas_call(
        _f3_kernel,
        grid=(NT,),
        in_specs=[pl.BlockSpec((TR, 256), lambda i: (i, 0)),
                  pl.BlockSpec((256, INT), lambda i: (0, 0)),
                  pl.BlockSpec((2, 256), lambda i: (0, 0))],
        out_specs=[pl.BlockSpec((TR, INT), lambda i: (i, 0)),
                   pl.BlockSpec((2, INT), lambda i: (0, 0))],
        out_shape=[jax.ShapeDtypeStruct((R, INT), f32),
                   jax.ShapeDtypeStruct((2, INT), f32)],
        compiler_params=seq,
    )(y2, W_nn2_3, s2)

    y3r = y3.reshape(S, N, B, INT)
    emb4 = pl.pallas_call(
        _f4_kernel,
        grid=(N,),
        in_specs=[pl.BlockSpec((S, 1, B, INT), lambda n: (0, n, 0, 0)),
                  pl.BlockSpec((2, INT), lambda n: (0, 0)),
                  pl.BlockSpec((1, INT, 256), lambda n: (n, 0, 0)),
                  pl.BlockSpec((1, 1, 256), lambda n: (n, 0, 0)),
                  pl.BlockSpec((1, 256, EMB), lambda n: (n, 0, 0)),
                  pl.BlockSpec((1, 1, EMB), lambda n: (n, 0, 0))],
        out_specs=pl.BlockSpec((S, 1, B, EMB), lambda n: (0, n, 0, 0)),
        out_shape=jax.ShapeDtypeStruct((S, N, B, EMB), f32),
        compiler_params=pltpu.CompilerParams(
            dimension_semantics=("arbitrary",)),
    )(y3r, s3, W_node1, b_node1[:, None, :], W_node2,
      b_node2[:, None, :])

    wxT = Wx.transpose(0, 2, 3, 1, 4).reshape(L, K * EMB, 4 * HID)
    whT = Wh.transpose(0, 2, 3, 1, 4).reshape(L, K * HID, 4 * HID)
    bsum = (bx + bh + bgate).reshape(L, 4 * HID)
    blin = b_lin.reshape(1, 1)

    out2 = pl.pallas_call(
        _lstm_kernel,
        grid=(S // U,),
        in_specs=[pl.BlockSpec((U, N, B, EMB), lambda t: (t, 0, 0, 0)),
                  pl.BlockSpec((2 * N, N), lambda t: (0, 0)),
                  pl.BlockSpec((L, K * EMB, 4 * HID), lambda t: (0, 0, 0)),
                  pl.BlockSpec((L, K * HID, 4 * HID), lambda t: (0, 0, 0)),
                  pl.BlockSpec((L, 4 * HID), lambda t: (0, 0)),
                  pl.BlockSpec((L, 3, HID), lambda t: (0, 0, 0)),
                  pl.BlockSpec((HID, 1), lambda t: (0, 0)),
                  pl.BlockSpec((1, 1), lambda t: (0, 0))],
        out_specs=pl.BlockSpec((NB, 1), lambda t: (0, 0)),
        out_shape=jax.ShapeDtypeStruct((NB, 1), f32),
        scratch_shapes=[pltpu.VMEM((L, NB, 3 * EMB), f32),
                        pltpu.VMEM((L, NB, EMB), f32)],
        compiler_params=pltpu.CompilerParams(
            dimension_semantics=("arbitrary",)),
    )(emb4, mcat, wxT, whT, bsum, wc, W_lin, blin)

    return out2.reshape(N, B, 1).transpose(1, 0, 2)
